# baseline (speedup 1.0000x reference)
import functools

import jax
import jax.numpy as jnp
from jax.experimental import pallas as pl

_G = 16


def _body(n_nodes,
          x_ref, a_ref,
          W0_ref, b0_ref, W1_ref, b1_ref, W2_ref, b2_ref,
          W3_ref, b3_ref, W4_ref, b4_ref,
          Wr_ref, br_ref, Wp1_ref, bp1_ref, Wp2_ref, bp2_ref,
          out_ref):
    G, N, F = x_ref.shape
    H = W0_ref.shape[1]
    xb = x_ref[...].reshape(G * N, F)
    h = jnp.dot(xb, W0_ref[...], preferred_element_type=jnp.float32)
    h = h + b0_ref[...]
    for W_ref, b_ref in ((W1_ref, b1_ref), (W2_ref, b2_ref),
                         (W3_ref, b3_ref), (W4_ref, b4_ref)):
        hw = jnp.dot(h, W_ref[...], preferred_element_type=jnp.float32)
        hw = hw.astype(jnp.bfloat16)
        parts = [
            jnp.dot(a_ref[g], hw[g * N:(g + 1) * N],
                    preferred_element_type=jnp.float32)
            for g in range(G)
        ]
        h = jnp.concatenate(parts, axis=0) + b_ref[...]
        h = jnp.maximum(h, 0.0)
    s = jnp.sum(h.reshape(G, N, H), axis=1)
    r = jnp.dot(s, Wr_ref[...], preferred_element_type=jnp.float32)
    r = r + n_nodes * br_ref[...]
    t = jnp.dot(r, Wp1_ref[...], preferred_element_type=jnp.float32)
    t = t + bp1_ref[...]
    t = jnp.where(t > 0, t, jnp.exp(t) - 1.0)
    o = jnp.dot(t, Wp2_ref[...], preferred_element_type=jnp.float32)
    out_ref[...] = jnp.broadcast_to(o, out_ref.shape) + bp2_ref[...]


def kernel(x, a, W0, b0, W1, b1, W2, b2, W3, b3, W4, b4,
           Wr, br, Wp1, bp1, Wp2, bp2):
    B, N, F = x.shape
    H = W0.shape[1]
    G = _G
    b0r = b0.reshape(1, H)
    b1r = b1.reshape(1, H)
    b2r = b2.reshape(1, H)
    b3r = b3.reshape(1, H)
    b4r = b4.reshape(1, H)
    brr = br.reshape(1, -1)
    bp1r = bp1.reshape(1, -1)
    bp2r = jnp.broadcast_to(bp2.reshape(1, 1), (1, 128))

    full = lambda arr: pl.BlockSpec(arr.shape, lambda i: (0,) * arr.ndim)
    out = pl.pallas_call(
        functools.partial(_body, float(N)),
        grid=(B // G,),
        in_specs=[
            pl.BlockSpec((G, N, F), lambda i: (i, 0, 0)),
            pl.BlockSpec((G, N, N), lambda i: (i, 0, 0)),
            full(W0), full(b0r), full(W1), full(b1r), full(W2), full(b2r),
            full(W3), full(b3r), full(W4), full(b4r),
            full(Wr), full(brr), full(Wp1), full(bp1r), full(Wp2), full(bp2r),
        ],
        out_specs=pl.BlockSpec((G, 128), lambda i: (i, 0)),
        out_shape=jax.ShapeDtypeStruct((B, 128), jnp.float32),
    )(x, a.astype(jnp.bfloat16), W0, b0r, W1, b1r, W2, b2r, W3, b3r, W4, b4r,
      Wr, brr, Wp1, bp1r, Wp2, bp2r)
    return out[:, 0]


# speedup vs baseline: 1.3632x; 1.3632x over previous
import functools

import jax
import jax.numpy as jnp
from jax.experimental import pallas as pl

_G = 16


def _body(n_nodes,
          x_ref, a_ref,
          W0_ref, b0_ref, W1_ref, b1_ref, W2_ref, b2_ref,
          W3_ref, b3_ref, W4_ref, b4_ref,
          Wr_ref, br_ref, Wp1_ref, bp1_ref, Wp2_ref, bp2_ref,
          out_ref):
    G, N, F = x_ref.shape
    H = W0_ref.shape[1]
    xb = x_ref[...].reshape(G * N, F)
    W01 = jnp.dot(W0_ref[...], W1_ref[...], preferred_element_type=jnp.float32)
    b01 = jnp.dot(b0_ref[...], W1_ref[...], preferred_element_type=jnp.float32)
    hw = jnp.dot(xb, W01, preferred_element_type=jnp.float32) + b01
    parts = [
        jnp.dot(a_ref[g], hw[g * N:(g + 1) * N],
                preferred_element_type=jnp.float32)
        for g in range(G)
    ]
    h = jnp.maximum(jnp.concatenate(parts, axis=0) + b1_ref[...], 0.0)
    for W_ref, b_ref in ((W2_ref, b2_ref), (W3_ref, b3_ref),
                         (W4_ref, b4_ref)):
        hw = jnp.dot(h, W_ref[...], preferred_element_type=jnp.float32)
        parts = [
            jnp.dot(a_ref[g], hw[g * N:(g + 1) * N],
                    preferred_element_type=jnp.float32)
            for g in range(G)
        ]
        h = jnp.concatenate(parts, axis=0) + b_ref[...]
        h = jnp.maximum(h, 0.0)
    s = jnp.sum(h.reshape(G, N, H), axis=1)
    r = jnp.dot(s, Wr_ref[...], preferred_element_type=jnp.float32)
    r = r + n_nodes * br_ref[...]
    t = jnp.dot(r, Wp1_ref[...], preferred_element_type=jnp.float32)
    t = t + bp1_ref[...]
    t = jnp.where(t > 0, t, jnp.exp(t) - 1.0)
    o = jnp.dot(t, Wp2_ref[...], preferred_element_type=jnp.float32)
    out_ref[...] = jnp.broadcast_to(o, out_ref.shape) + bp2_ref[...]


def kernel(x, a, W0, b0, W1, b1, W2, b2, W3, b3, W4, b4,
           Wr, br, Wp1, bp1, Wp2, bp2):
    B, N, F = x.shape
    H = W0.shape[1]
    G = _G
    b0r = b0.reshape(1, H)
    b1r = b1.reshape(1, H)
    b2r = b2.reshape(1, H)
    b3r = b3.reshape(1, H)
    b4r = b4.reshape(1, H)
    brr = br.reshape(1, -1)
    bp1r = bp1.reshape(1, -1)
    bp2r = jnp.broadcast_to(bp2.reshape(1, 1), (1, 128))

    full = lambda arr: pl.BlockSpec(arr.shape, lambda i: (0,) * arr.ndim)
    out = pl.pallas_call(
        functools.partial(_body, float(N)),
        grid=(B // G,),
        in_specs=[
            pl.BlockSpec((G, N, F), lambda i: (i, 0, 0)),
            pl.BlockSpec((G, N, N), lambda i: (i, 0, 0)),
            full(W0), full(b0r), full(W1), full(b1r), full(W2), full(b2r),
            full(W3), full(b3r), full(W4), full(b4r),
            full(Wr), full(brr), full(Wp1), full(bp1r), full(Wp2), full(bp2r),
        ],
        out_specs=pl.BlockSpec((G, 128), lambda i: (i, 0)),
        out_shape=jax.ShapeDtypeStruct((B, 128), jnp.float32),
    )(x, a, W0, b0r, W1, b1r, W2, b2r, W3, b3r, W4, b4r,
      Wr, brr, Wp1, bp1r, Wp2, bp2r)
    return out[:, 0]


# speedup vs baseline: 1.4196x; 1.0414x over previous
import functools

import jax
import jax.numpy as jnp
from jax.experimental import pallas as pl

_G = 16


def _body(n_nodes,
          x_ref, a_ref,
          W0_ref, b0_ref, W1_ref, b1_ref, W2_ref, b2_ref,
          W3_ref, b3_ref, W4_ref, b4_ref,
          Wr_ref, br_ref, Wp1_ref, bp1_ref, Wp2_ref, bp2_ref,
          out_ref):
    G, N, F = x_ref.shape
    H = W0_ref.shape[1]
    xb = x_ref[...].reshape(G * N, F)
    W01 = jnp.dot(W0_ref[...], W1_ref[...], preferred_element_type=jnp.float32)
    b01 = jnp.dot(b0_ref[...], W1_ref[...], preferred_element_type=jnp.float32)
    hw = jnp.dot(xb, W01, preferred_element_type=jnp.float32) + b01
    parts = [
        jnp.dot(a_ref[g], hw[g * N:(g + 1) * N],
                preferred_element_type=jnp.float32)
        for g in range(G)
    ]
    h = jnp.maximum(jnp.concatenate(parts, axis=0) + b1_ref[...], 0.0)
    for W_ref, b_ref in ((W2_ref, b2_ref), (W3_ref, b3_ref),
                         (W4_ref, b4_ref)):
        hw = jnp.dot(h, W_ref[...], preferred_element_type=jnp.float32)
        parts = [
            jnp.dot(a_ref[g], hw[g * N:(g + 1) * N],
                    preferred_element_type=jnp.float32)
            for g in range(G)
        ]
        h = jnp.concatenate(parts, axis=0) + b_ref[...]
        h = jnp.maximum(h, 0.0)
    s = jnp.sum(h.reshape(G, N, H), axis=1)
    r = jnp.dot(s, Wr_ref[...], preferred_element_type=jnp.float32)
    r = r + n_nodes * br_ref[...]
    t = jnp.dot(r, Wp1_ref[...], preferred_element_type=jnp.float32)
    t = t + bp1_ref[...]
    t = jnp.where(t > 0, t, jnp.exp(t) - 1.0)
    o = jnp.dot(t, Wp2_ref[...], preferred_element_type=jnp.float32)
    out_ref[...] = o + bp2_ref[...]


def kernel(x, a, W0, b0, W1, b1, W2, b2, W3, b3, W4, b4,
           Wr, br, Wp1, bp1, Wp2, bp2):
    B, N, F = x.shape
    H = W0.shape[1]
    G = _G
    b0r = b0.reshape(1, H)
    b1r = b1.reshape(1, H)
    b2r = b2.reshape(1, H)
    b3r = b3.reshape(1, H)
    b4r = b4.reshape(1, H)
    brr = br.reshape(1, -1)
    bp1r = bp1.reshape(1, -1)
    bp2r = bp2.reshape(1, 1)

    full = lambda arr: pl.BlockSpec(arr.shape, lambda i: (0,) * arr.ndim)
    out = pl.pallas_call(
        functools.partial(_body, float(N)),
        grid=(B // G,),
        in_specs=[
            pl.BlockSpec((G, N, F), lambda i: (i, 0, 0)),
            pl.BlockSpec((G, N, N), lambda i: (i, 0, 0)),
            full(W0), full(b0r), full(W1), full(b1r), full(W2), full(b2r),
            full(W3), full(b3r), full(W4), full(b4r),
            full(Wr), full(brr), full(Wp1), full(bp1r), full(Wp2), full(bp2r),
        ],
        out_specs=pl.BlockSpec((G, 1), lambda i: (i, 0)),
        out_shape=jax.ShapeDtypeStruct((B, 1), jnp.float32),
    )(x, a, W0, b0r, W1, b1r, W2, b2r, W3, b3r, W4, b4r,
      Wr, brr, Wp1, bp1r, Wp2, bp2r)
    return out.reshape(B)


# speedup vs baseline: 1.5169x; 1.0686x over previous
import functools

import jax
import jax.numpy as jnp
from jax.experimental import pallas as pl

_G = 16


def _body(n_nodes,
          x_ref, a_ref,
          W0_ref, b0_ref, W1_ref, b1_ref, W2_ref, b2_ref,
          W3_ref, b3_ref, W4_ref, b4_ref,
          Wr_ref, br_ref, Wp1t_ref, bp1_ref, Wp2_ref, bp2_ref,
          out_ref):
    G, N, F = x_ref.shape
    H = W0_ref.shape[1]
    b0r = b0_ref[...].reshape(1, H)
    b1r = b1_ref[...].reshape(1, H)
    b2r = b2_ref[...].reshape(1, H)
    b3r = b3_ref[...].reshape(1, H)
    b4r = b4_ref[...].reshape(1, H)
    xb = x_ref[...].reshape(G * N, F)
    W01 = jnp.dot(W0_ref[...], W1_ref[...], preferred_element_type=jnp.float32)
    b01 = jnp.dot(b0r, W1_ref[...], preferred_element_type=jnp.float32)
    hw = jnp.dot(xb, W01, preferred_element_type=jnp.float32) + b01
    parts = [
        jnp.dot(a_ref[g], hw[g * N:(g + 1) * N],
                preferred_element_type=jnp.float32)
        for g in range(G)
    ]
    h = jnp.maximum(jnp.concatenate(parts, axis=0) + b1r, 0.0)
    for W_ref, br_ in ((W2_ref, b2r), (W3_ref, b3r), (W4_ref, b4r)):
        hw = jnp.dot(h, W_ref[...], preferred_element_type=jnp.float32)
        parts = [
            jnp.dot(a_ref[g], hw[g * N:(g + 1) * N],
                    preferred_element_type=jnp.float32)
            for g in range(G)
        ]
        h = jnp.concatenate(parts, axis=0) + br_
        h = jnp.maximum(h, 0.0)
    s = jnp.sum(h.reshape(G, N, H), axis=1)
    r = jnp.dot(s, Wr_ref[...], preferred_element_type=jnp.float32)
    r = r + n_nodes * br_ref[...].reshape(1, -1)
    t = jax.lax.dot_general(r, Wp1t_ref[...], (((1,), (1,)), ((), ())),
                            preferred_element_type=jnp.float32)
    t = t + bp1_ref[...].reshape(1, -1)
    t = jnp.where(t > 0, t, jnp.exp(t) - 1.0)
    o = jax.lax.dot_general(Wp2_ref[...], t, (((1,), (1,)), ((), ())),
                            preferred_element_type=jnp.float32)
    out_ref[...] = (o + bp2_ref[...].reshape(1, 1)).reshape(1, 1, G)


def kernel(x, a, W0, b0, W1, b1, W2, b2, W3, b3, W4, b4,
           Wr, br, Wp1, bp1, Wp2, bp2):
    B, N, F = x.shape
    H = W0.shape[1]
    G = _G
    full = lambda arr: pl.BlockSpec(arr.shape, lambda i: (0,) * arr.ndim)
    out = pl.pallas_call(
        functools.partial(_body, float(N)),
        grid=(B // G,),
        in_specs=[
            pl.BlockSpec((G, N, F), lambda i: (i, 0, 0)),
            pl.BlockSpec((G, N, N), lambda i: (i, 0, 0)),
            full(W0), full(b0), full(W1), full(b1), full(W2), full(b2),
            full(W3), full(b3), full(W4), full(b4),
            full(Wr), full(br), full(Wp1.T), full(bp1),
            full(Wp2.reshape(1, -1)), full(bp2),
        ],
        out_specs=pl.BlockSpec((1, 1, G), lambda i: (i, 0, 0)),
        out_shape=jax.ShapeDtypeStruct((B // G, 1, G), jnp.float32),
    )(x, a, W0, b0, W1, b1, W2, b2, W3, b3, W4, b4,
      Wr, br, Wp1.T, bp1, Wp2.reshape(1, -1), bp2)
    return out.reshape(B)


# speedup vs baseline: 1.8064x; 1.1908x over previous
import functools

import jax
import jax.numpy as jnp
from jax.experimental import pallas as pl
from jax.experimental.pallas import tpu as pltpu

_G = 16


def _body(n_nodes, n_steps,
          x_ref, a_ref,
          W0_ref, b0_ref, W1_ref, b1_ref, W2_ref, b2_ref,
          W3_ref, b3_ref, W4_ref, b4_ref,
          Wr_ref, br_ref, Wp1t_ref, bp1_ref, Wp2_ref, bp2_ref,
          out_ref, W01_ref, b01_ref, S_ref):
    G, N, F = x_ref.shape
    H = W0_ref.shape[1]
    i = pl.program_id(0)
    b0r = b0_ref[...].reshape(1, H)
    b1r = b1_ref[...].reshape(1, H)
    b2r = b2_ref[...].reshape(1, H)
    b3r = b3_ref[...].reshape(1, H)
    b4r = b4_ref[...].reshape(1, H)

    @pl.when(i == 0)
    def _fold():
        W01_ref[...] = jnp.dot(W0_ref[...], W1_ref[...],
                               preferred_element_type=jnp.float32)
        b01_ref[...] = jnp.dot(b0r, W1_ref[...],
                               preferred_element_type=jnp.float32)

    xb = x_ref[...].reshape(G * N, F)
    hw = jnp.dot(xb, W01_ref[...], preferred_element_type=jnp.float32)
    hw = hw + b01_ref[...]
    parts = [
        jnp.dot(a_ref[g], hw[g * N:(g + 1) * N],
                preferred_element_type=jnp.float32)
        for g in range(G)
    ]
    h = jnp.maximum(jnp.concatenate(parts, axis=0) + b1r, 0.0)
    for W_ref, br_ in ((W2_ref, b2r), (W3_ref, b3r), (W4_ref, b4r)):
        hw = jnp.dot(h, W_ref[...], preferred_element_type=jnp.float32)
        parts = [
            jnp.dot(a_ref[g], hw[g * N:(g + 1) * N],
                    preferred_element_type=jnp.float32)
            for g in range(G)
        ]
        h = jnp.concatenate(parts, axis=0) + br_
        h = jnp.maximum(h, 0.0)
    S_ref[pl.ds(i * G, G), :] = jnp.sum(h.reshape(G, N, H), axis=1)

    @pl.when(i == n_steps - 1)
    def _head():
        s = S_ref[...]
        r = jnp.dot(s, Wr_ref[...], preferred_element_type=jnp.float32)
        r = r + n_nodes * br_ref[...].reshape(1, -1)
        t = jax.lax.dot_general(r, Wp1t_ref[...], (((1,), (1,)), ((), ())),
                                preferred_element_type=jnp.float32)
        t = t + bp1_ref[...].reshape(1, -1)
        t = jnp.where(t > 0, t, jnp.exp(t) - 1.0)
        o = jax.lax.dot_general(Wp2_ref[...], t, (((1,), (1,)), ((), ())),
                                preferred_element_type=jnp.float32)
        out_ref[...] = o + bp2_ref[...].reshape(1, 1)


def kernel(x, a, W0, b0, W1, b1, W2, b2, W3, b3, W4, b4,
           Wr, br, Wp1, bp1, Wp2, bp2):
    B, N, F = x.shape
    H = W0.shape[1]
    G = _G
    n_steps = B // G
    full = lambda arr: pl.BlockSpec(arr.shape, lambda i: (0,) * arr.ndim)
    out = pl.pallas_call(
        functools.partial(_body, float(N), n_steps),
        grid=(n_steps,),
        in_specs=[
            pl.BlockSpec((G, N, F), lambda i: (i, 0, 0)),
            pl.BlockSpec((G, N, N), lambda i: (i, 0, 0)),
            full(W0), full(b0), full(W1), full(b1), full(W2), full(b2),
            full(W3), full(b3), full(W4), full(b4),
            full(Wr), full(br), full(Wp1.T), full(bp1),
            full(Wp2.reshape(1, -1)), full(bp2),
        ],
        out_specs=pl.BlockSpec((1, B), lambda i: (0, 0)),
        out_shape=jax.ShapeDtypeStruct((1, B), jnp.float32),
        scratch_shapes=[
            pltpu.VMEM((H, H), jnp.float32),
            pltpu.VMEM((1, H), jnp.float32),
            pltpu.VMEM((B, H), jnp.float32),
        ],
    )(x, a, W0, b0, W1, b1, W2, b2, W3, b3, W4, b4,
      Wr, br, Wp1.T, bp1, Wp2.reshape(1, -1), bp2)
    return out.reshape(B)
